# baseline (speedup 1.0000x reference)
import functools

import jax
import jax.numpy as jnp
from jax import lax
from jax.experimental import pallas as pl
from jax.experimental.pallas import tpu as pltpu
from jax.experimental.pallas import tpu_sc as plsc

NC = 2
NS = 16
NW = NC * NS
CH = 80
L = 16


def _proj_body(x_ref, ws_ref, wd_ref, a_ref, sx_ref, tx_ref, s_ref, t_ref):
    x = x_ref[...]
    sx = jnp.dot(x, ws_ref[...], preferred_element_type=jnp.float32)
    tx = jnp.dot(x, wd_ref[...], preferred_element_type=jnp.float32)
    sx_ref[...] = sx
    tx_ref[...] = tx
    a = a_ref[...]
    d = a.shape[0] // 2
    s_ref[...] = jnp.dot(sx, a[:d], preferred_element_type=jnp.float32)
    t_ref[...] = jnp.dot(tx, a[d:], preferred_element_type=jnp.float32)


def _epi_body(tx_ref, agg_ref, div_ref, out_ref):
    agg = agg_ref[0] + agg_ref[1]
    d = jnp.sum(div_ref[...], axis=1, keepdims=True)
    out_ref[...] = tx_ref[...] + agg / (d + 1e-6)


NRB = 2
NIB = 4


def _make_edge_kernel(n, out_c, e):
    epw = e // NW
    nchunk = epw // CH
    npt = (n // NS) // 8 * 8
    rem = n - NS * npt
    unroll = 4
    npairs = (nchunk - 1) // unroll
    assert npairs * unroll + 1 == nchunk
    mesh = plsc.VectorSubcoreMesh(core_axis_name="c", subcore_axis_name="s")

    @functools.partial(
        pl.kernel,
        mesh=mesh,
        compiler_params=pltpu.CompilerParams(needs_layout_passes=False),
        out_type=[
            jax.ShapeDtypeStruct((NC, n, out_c), jnp.float32),
            jax.ShapeDtypeStruct((NC, n), jnp.float32),
        ],
        scratch_types=(
            [
                pltpu.VMEM((n,), jnp.float32),
                pltpu.VMEM((n,), jnp.float32),
                pltpu.VMEM_SHARED((n, out_c), jnp.float32),
                pltpu.VMEM_SHARED((n,), jnp.float32),
            ]
            + [pltpu.VMEM((CH, out_c), jnp.float32) for _ in range(NRB)]
            + [pltpu.VMEM((CH,), jnp.float32) for _ in range(NRB)]
            + [pltpu.VMEM((CH,), jnp.int32) for _ in range(2 * NIB)]
            + [pltpu.SemaphoreType.DMA for _ in range(3 * NRB + NIB)]
        ),
    )
    def edge_kernel(srcx_hbm, s_hbm, t_hbm, si_hbm, di_hbm, z_hbm, zd_hbm,
                    agg_out, div_out, s_v, t_v, agg_sh, div_sh, *bufs):
        rows = bufs[:NRB]
        att = bufs[NRB:2 * NRB]
        si = bufs[2 * NRB:2 * NRB + NIB]
        di = bufs[2 * NRB + NIB:2 * NRB + 2 * NIB]
        sems = bufs[2 * NRB + 2 * NIB:]
        gsem = sems[:NRB]
        ssem = sems[NRB:2 * NRB]
        dsem = sems[2 * NRB:3 * NRB]
        isem = sems[3 * NRB:]
        cid = lax.axis_index("c")
        sid = lax.axis_index("s")
        wid = sid * NC + cid

        pltpu.sync_copy(z_hbm, agg_sh.at[pl.ds(sid * npt, npt)])
        if rem:
            @pl.when(sid == 0)
            def _():
                pltpu.sync_copy(z_hbm.at[pl.ds(0, rem)],
                                agg_sh.at[pl.ds(NS * npt, rem)])

        @pl.when(sid == 0)
        def _():
            pltpu.sync_copy(zd_hbm, div_sh)

        pltpu.sync_copy(s_hbm, s_v)
        pltpu.sync_copy(t_hbm, t_v)

        plsc.subcore_barrier()

        def _idx_start(c, i):
            base = wid * epw + c * CH
            pltpu.async_copy(si_hbm.at[pl.ds(base, CH)], si[i], isem[i])
            pltpu.async_copy(di_hbm.at[pl.ds(base, CH)], di[i], isem[i])

        def _idx_wait(c, i):
            base = wid * epw + c * CH
            pltpu.make_async_copy(si_hbm.at[pl.ds(base, CH)], si[i],
                                  isem[i]).wait()
            pltpu.make_async_copy(di_hbm.at[pl.ds(base, CH)], di[i],
                                  isem[i]).wait()

        def _chunk(c, b, i):
            nb = 1 - b
            pltpu.make_async_copy(srcx_hbm.at[si[i]], rows[b], gsem[b]).wait()

            for g in range(CH // L):
                sidx = si[i][pl.ds(g * L, L)]
                didx = di[i][pl.ds(g * L, L)]
                z = plsc.load_gather(s_v, [sidx]) + plsc.load_gather(t_v, [didx])
                att[b][pl.ds(g * L, L)] = jnp.exp(jnp.where(z >= 0.0, z, 0.2 * z))

            cc = c
            @pl.when(cc >= 1)
            def _():
                pltpu.make_async_copy(
                    rows[nb], agg_sh.at[di[(i - 1) % NIB]], ssem[nb]).wait()
                pltpu.make_async_copy(
                    att[nb], div_sh.at[di[(i - 1) % NIB]], dsem[nb]).wait()

            @pl.when(cc + 1 < nchunk)
            def _():
                _idx_wait(cc + 1, (i + 1) % NIB)
                pltpu.async_copy(srcx_hbm.at[si[(i + 1) % NIB]], rows[nb],
                                 gsem[nb])

            @plsc.parallel_loop(0, CH, 1, unroll=8)
            def _scale(r):
                av = plsc.load_gather(att[b], [jnp.zeros((L,), jnp.int32) + r])
                for j in range(out_c // L):
                    sl = pl.ds(j * L, L)
                    rows[b][r, sl] = rows[b][r, sl] * av

            pltpu.async_copy(rows[b], agg_sh.at[di[i]], ssem[b], add=True)
            pltpu.async_copy(att[b], div_sh.at[di[i]], dsem[b], add=True)

            @pl.when(cc + 2 < nchunk)
            def _():
                _idx_start(cc + 2, (i + 2) % NIB)

        base0 = wid * epw
        pltpu.sync_copy(si_hbm.at[pl.ds(base0, CH)], si[0])
        pltpu.sync_copy(di_hbm.at[pl.ds(base0, CH)], di[0])
        pltpu.async_copy(srcx_hbm.at[si[0]], rows[0], gsem[0])
        _idx_start(jnp.int32(1), 1)

        def _outer(o, carry):
            for k in range(unroll):
                _chunk(o * unroll + k, k % 2, k)
            return carry

        lax.fori_loop(0, npairs, _outer, 0)
        _chunk(jnp.int32(nchunk - 1), 0, 0)

        pltpu.make_async_copy(rows[0], agg_sh.at[di[0]], ssem[0]).wait()
        pltpu.make_async_copy(att[0], div_sh.at[di[0]], dsem[0]).wait()

        plsc.subcore_barrier()

        sl = pl.ds(sid * npt, npt)
        pltpu.sync_copy(agg_sh.at[sl], agg_out.at[cid, sl])
        if rem:
            @pl.when(sid == 0)
            def _():
                slr = pl.ds(NS * npt, rem)
                pltpu.sync_copy(agg_sh.at[slr], agg_out.at[cid, slr])
        @pl.when(sid == 0)
        def _():
            pltpu.sync_copy(div_sh, div_out.at[cid])

    return edge_kernel


def kernel(x, edge_index, W_src, W_dst, a):
    n, d = x.shape
    out_c = W_src.shape[1]
    e = edge_index.shape[1]

    blk = 1000
    grid = (n // blk,)
    source_x, target_x, s2, t2 = pl.pallas_call(
        _proj_body,
        grid=grid,
        in_specs=[
            pl.BlockSpec((blk, d), lambda i: (i, 0)),
            pl.BlockSpec((d, out_c), lambda i: (0, 0)),
            pl.BlockSpec((d, out_c), lambda i: (0, 0)),
            pl.BlockSpec((2 * out_c, 1), lambda i: (0, 0)),
        ],
        out_specs=[
            pl.BlockSpec((blk, out_c), lambda i: (i, 0)),
            pl.BlockSpec((blk, out_c), lambda i: (i, 0)),
            pl.BlockSpec((blk, 1), lambda i: (i, 0)),
            pl.BlockSpec((blk, 1), lambda i: (i, 0)),
        ],
        out_shape=[
            jax.ShapeDtypeStruct((n, out_c), jnp.float32),
            jax.ShapeDtypeStruct((n, out_c), jnp.float32),
            jax.ShapeDtypeStruct((n, 1), jnp.float32),
            jax.ShapeDtypeStruct((n, 1), jnp.float32),
        ],
    )(x, W_src, W_dst, a)

    s = s2.reshape(n)
    t = t2.reshape(n)
    si = edge_index[0].reshape(e)
    di = edge_index[1].reshape(e)
    zeros_blk = jnp.zeros(((n // NS) // 8 * 8, out_c), jnp.float32)
    zeros_div = jnp.zeros((n,), jnp.float32)

    edge_kernel = _make_edge_kernel(n, out_c, e)
    agg_pair, div_parts = edge_kernel(source_x, s, t, si, di,
                                      zeros_blk, zeros_div)

    out = pl.pallas_call(
        _epi_body,
        grid=grid,
        in_specs=[
            pl.BlockSpec((blk, out_c), lambda i: (i, 0)),
            pl.BlockSpec((NC, blk, out_c), lambda i: (0, i, 0)),
            pl.BlockSpec((blk, NC), lambda i: (i, 0)),
        ],
        out_specs=pl.BlockSpec((blk, out_c), lambda i: (i, 0)),
        out_shape=jax.ShapeDtypeStruct((n, out_c), jnp.float32),
    )(target_x, agg_pair, div_parts.T)
    return out


# speedup vs baseline: 1.1852x; 1.1852x over previous
import functools

import jax
import jax.numpy as jnp
from jax import lax
from jax.experimental import pallas as pl
from jax.experimental.pallas import tpu as pltpu
from jax.experimental.pallas import tpu_sc as plsc

NC = 2
NS = 16
NW = NC * NS
CH = 80
L = 16


def _proj_body(x_ref, ws_ref, wd_ref, a_ref, sx_ref, tx_ref, s_ref, t_ref):
    x = x_ref[...]
    sx = jnp.dot(x, ws_ref[...], preferred_element_type=jnp.float32)
    tx = jnp.dot(x, wd_ref[...], preferred_element_type=jnp.float32)
    sx_ref[...] = sx
    tx_ref[...] = tx
    a = a_ref[...]
    d = a.shape[0] // 2
    s_ref[...] = jnp.dot(sx, a[:d], preferred_element_type=jnp.float32)
    t_ref[...] = jnp.dot(tx, a[d:], preferred_element_type=jnp.float32)


def _epi_body(tx_ref, agg_ref, div_ref, out_ref):
    agg = agg_ref[0] + agg_ref[1]
    d = jnp.sum(div_ref[...], axis=1, keepdims=True)
    out_ref[...] = tx_ref[...] + agg / (d + 1e-6)


NRB = 4
NIB = 8


def _make_edge_kernel(n, out_c, e):
    epw = e // NW
    nchunk = epw // CH
    npt = (n // NS) // 8 * 8
    rem = n - NS * npt
    unroll = NIB
    peel = nchunk % unroll
    nouter = nchunk // unroll
    assert peel < NRB + 2
    mesh = plsc.VectorSubcoreMesh(core_axis_name="c", subcore_axis_name="s")

    @functools.partial(
        pl.kernel,
        mesh=mesh,
        compiler_params=pltpu.CompilerParams(needs_layout_passes=False),
        out_type=[
            jax.ShapeDtypeStruct((NC, n, out_c), jnp.float32),
            jax.ShapeDtypeStruct((NC, n), jnp.float32),
        ],
        scratch_types=(
            [
                pltpu.VMEM_SHARED((n, out_c), jnp.float32),
                pltpu.VMEM_SHARED((n,), jnp.float32),
            ]
            + [pltpu.VMEM((CH, out_c), jnp.float32) for _ in range(NRB)]
            + [pltpu.VMEM((CH,), jnp.float32) for _ in range(3 * NRB)]
            + [pltpu.VMEM((CH,), jnp.int32) for _ in range(2 * NIB)]
            + [pltpu.SemaphoreType.DMA for _ in range(3 * NRB + NIB)]
        ),
    )
    def edge_kernel(srcx_hbm, s_hbm, t_hbm, si_hbm, di_hbm, z_hbm, zd_hbm,
                    agg_out, div_out, agg_sh, div_sh, *bufs):
        rows = bufs[:NRB]
        sg = bufs[NRB:2 * NRB]
        tg = bufs[2 * NRB:3 * NRB]
        att = bufs[3 * NRB:4 * NRB]
        si = bufs[4 * NRB:4 * NRB + NIB]
        di = bufs[4 * NRB + NIB:4 * NRB + 2 * NIB]
        sems = bufs[4 * NRB + 2 * NIB:]
        gsem = sems[:NRB]
        ssem = sems[NRB:2 * NRB]
        dsem = sems[2 * NRB:3 * NRB]
        isem = sems[3 * NRB:]
        cid = lax.axis_index("c")
        sid = lax.axis_index("s")
        wid = sid * NC + cid

        pltpu.sync_copy(z_hbm, agg_sh.at[pl.ds(sid * npt, npt)])
        if rem:
            @pl.when(sid == 0)
            def _():
                pltpu.sync_copy(z_hbm.at[pl.ds(0, rem)],
                                agg_sh.at[pl.ds(NS * npt, rem)])

        @pl.when(sid == 0)
        def _():
            pltpu.sync_copy(zd_hbm, div_sh)

        plsc.subcore_barrier()

        def _idx_start(c, i):
            base = wid * epw + c * CH
            pltpu.async_copy(si_hbm.at[pl.ds(base, CH)], si[i], isem[i])
            pltpu.async_copy(di_hbm.at[pl.ds(base, CH)], di[i], isem[i])

        def _gather_start(i, b):
            pltpu.async_copy(srcx_hbm.at[si[i]], rows[b], gsem[b])
            pltpu.async_copy(s_hbm.at[si[i]], sg[b], gsem[b])
            pltpu.async_copy(t_hbm.at[di[i]], tg[b], gsem[b])

        def _gather_wait(i, b):
            pltpu.make_async_copy(srcx_hbm.at[si[i]], rows[b], gsem[b]).wait()
            pltpu.make_async_copy(s_hbm.at[si[i]], sg[b], gsem[b]).wait()
            pltpu.make_async_copy(t_hbm.at[di[i]], tg[b], gsem[b]).wait()

        def _scatter_wait(b, i):
            pltpu.make_async_copy(rows[b], agg_sh.at[di[i]], ssem[b]).wait()
            pltpu.make_async_copy(att[b], div_sh.at[di[i]], dsem[b]).wait()

        def _chunk(c, b, i):
            _gather_wait(i, b)

            for g in range(CH // L):
                sl = pl.ds(g * L, L)
                z = sg[b][sl] + tg[b][sl]
                att[b][sl] = jnp.exp(jnp.where(z >= 0.0, z, 0.2 * z))

            cc = c
            @pl.when(cc >= 2)
            def _():
                _scatter_wait((b + 2) % NRB, (i + 6) % NIB)

            @pl.when(cc + 2 < nchunk)
            def _():
                i2 = (i + 2) % NIB
                base = wid * epw + (cc + 2) * CH
                pltpu.make_async_copy(si_hbm.at[pl.ds(base, CH)], si[i2],
                                      isem[i2]).wait()
                pltpu.make_async_copy(di_hbm.at[pl.ds(base, CH)], di[i2],
                                      isem[i2]).wait()
                _gather_start(i2, (b + 2) % NRB)

            @plsc.parallel_loop(0, CH, 1, unroll=8)
            def _scale(r):
                av = plsc.load_gather(att[b], [jnp.zeros((L,), jnp.int32) + r])
                for j in range(out_c // L):
                    sl = pl.ds(j * L, L)
                    rows[b][r, sl] = rows[b][r, sl] * av

            pltpu.async_copy(rows[b], agg_sh.at[di[i]], ssem[b], add=True)
            pltpu.async_copy(att[b], div_sh.at[di[i]], dsem[b], add=True)

            @pl.when(cc + 4 < nchunk)
            def _():
                _idx_start(cc + 4, (i + 4) % NIB)

        for k in range(2):
            base = wid * epw + k * CH
            pltpu.sync_copy(si_hbm.at[pl.ds(base, CH)], si[k])
            pltpu.sync_copy(di_hbm.at[pl.ds(base, CH)], di[k])
        _idx_start(jnp.int32(2), 2)
        _idx_start(jnp.int32(3), 3)
        _gather_start(0, 0)
        _gather_start(1, 1)

        def _outer(o, carry):
            for k in range(unroll):
                _chunk(o * unroll + k, k % NRB, k)
            return carry

        lax.fori_loop(0, nouter, _outer, 0)
        for k in range(peel):
            _chunk(jnp.int32(nouter * unroll + k), k % NRB, k)

        _scatter_wait((peel - 2) % NRB, (peel - 2) % NIB)
        _scatter_wait((peel - 1) % NRB, (peel - 1) % NIB)

        plsc.subcore_barrier()

        sl = pl.ds(sid * npt, npt)
        pltpu.sync_copy(agg_sh.at[sl], agg_out.at[cid, sl])
        if rem:
            @pl.when(sid == 0)
            def _():
                slr = pl.ds(NS * npt, rem)
                pltpu.sync_copy(agg_sh.at[slr], agg_out.at[cid, slr])
        @pl.when(sid == 0)
        def _():
            pltpu.sync_copy(div_sh, div_out.at[cid])

    return edge_kernel


def kernel(x, edge_index, W_src, W_dst, a):
    n, d = x.shape
    out_c = W_src.shape[1]
    e = edge_index.shape[1]

    blk = 1000
    grid = (n // blk,)
    source_x, target_x, s2, t2 = pl.pallas_call(
        _proj_body,
        grid=grid,
        in_specs=[
            pl.BlockSpec((blk, d), lambda i: (i, 0)),
            pl.BlockSpec((d, out_c), lambda i: (0, 0)),
            pl.BlockSpec((d, out_c), lambda i: (0, 0)),
            pl.BlockSpec((2 * out_c, 1), lambda i: (0, 0)),
        ],
        out_specs=[
            pl.BlockSpec((blk, out_c), lambda i: (i, 0)),
            pl.BlockSpec((blk, out_c), lambda i: (i, 0)),
            pl.BlockSpec((blk, 1), lambda i: (i, 0)),
            pl.BlockSpec((blk, 1), lambda i: (i, 0)),
        ],
        out_shape=[
            jax.ShapeDtypeStruct((n, out_c), jnp.float32),
            jax.ShapeDtypeStruct((n, out_c), jnp.float32),
            jax.ShapeDtypeStruct((n, 1), jnp.float32),
            jax.ShapeDtypeStruct((n, 1), jnp.float32),
        ],
    )(x, W_src, W_dst, a)

    s = s2.reshape(n)
    t = t2.reshape(n)
    si = edge_index[0].reshape(e)
    di = edge_index[1].reshape(e)
    zeros_blk = jnp.zeros(((n // NS) // 8 * 8, out_c), jnp.float32)
    zeros_div = jnp.zeros((n,), jnp.float32)

    edge_kernel = _make_edge_kernel(n, out_c, e)
    agg_pair, div_parts = edge_kernel(source_x, s, t, si, di,
                                      zeros_blk, zeros_div)

    out = pl.pallas_call(
        _epi_body,
        grid=grid,
        in_specs=[
            pl.BlockSpec((blk, out_c), lambda i: (i, 0)),
            pl.BlockSpec((NC, blk, out_c), lambda i: (0, i, 0)),
            pl.BlockSpec((blk, NC), lambda i: (i, 0)),
        ],
        out_specs=pl.BlockSpec((blk, out_c), lambda i: (i, 0)),
        out_shape=jax.ShapeDtypeStruct((n, out_c), jnp.float32),
    )(target_x, agg_pair, div_parts.T)
    return out
